# Initial kernel scaffold; baseline (speedup 1.0000x reference)
import jax
import jax.numpy as jnp
from jax.experimental import pallas as pl


def kernel(x, table):
    raise NotImplementedError("write your pallas kernel here")


# speedup vs baseline: 1.8758x; 1.8758x over previous
import jax
import jax.numpy as jnp
from jax import lax
from jax.experimental import pallas as pl
from jax.experimental.pallas import tpu as pltpu
from jax.experimental.pallas import tpu_sc as plsc

_W = 128
_NBUF = 4
_M = 2 * _NBUF
_NC = 2
_NS = 16
_NWORKERS = _NC * _NS


def _embedding_gather(flat_idx, table, num_indices, d_model):
    per_worker = num_indices // _NWORKERS
    nwin = per_worker // _W
    mesh = plsc.VectorSubcoreMesh(core_axis_name="core",
                                  subcore_axis_name="subcore")

    @pl.kernel(
        out_type=jax.ShapeDtypeStruct((num_indices, d_model), table.dtype),
        mesh=mesh,
        scratch_types=[
            pltpu.VMEM((per_worker,), jnp.int32),
            pltpu.VMEM((_M, _W, d_model), table.dtype),
            pltpu.SemaphoreType.DMA((_M,)),
            pltpu.SemaphoreType.DMA((_M,)),
        ],
        compiler_params=pltpu.CompilerParams(use_tc_tiling_on_sc=False),
    )
    def gather_kernel(table_hbm, idx_hbm, out_hbm, idx_v, rows_v, gsem, ssem):
        wid = lax.axis_index("subcore") * _NC + lax.axis_index("core")
        base = wid * per_worker
        pltpu.sync_copy(idx_hbm.at[pl.ds(base, per_worker)], idx_v)

        def gather(w, slot):
            return pltpu.make_async_copy(
                table_hbm.at[idx_v.at[pl.ds(w * _W, _W)]],
                rows_v.at[slot],
                gsem.at[slot],
            )

        def store(w, slot):
            return pltpu.make_async_copy(
                rows_v.at[slot],
                out_hbm.at[pl.ds(base + w * _W, _W)],
                ssem.at[slot],
            )

        for w in range(_NBUF):
            gather(w, w).start()

        @pl.loop(0, nwin, step=_M)
        def _(g):
            for j in range(_M):
                w = g + j
                gather(w, j).wait()
                store(w, j).start()
                v = w + _NBUF
                slot = (j + _NBUF) % _M

                @pl.when(jnp.logical_and(v < nwin, v >= _M))
                def _():
                    store(v - _M, slot).wait()

                @pl.when(v < nwin)
                def _():
                    gather(v, slot).start()

        for j in range(_M):
            store(0, j).wait()

    return gather_kernel(table, flat_idx)


def kernel(x, table):
    batch, hist = x.shape
    vocab, d_model = table.shape
    num_indices = batch * hist
    flat_idx = x.reshape(num_indices).astype(jnp.int32)
    out = _embedding_gather(flat_idx, table, num_indices, d_model)
    return out.reshape(batch, hist, d_model)
